# baseline (speedup 1.0000x reference)
import jax
import jax.numpy as jnp
from jax.experimental import pallas as pl
from jax.experimental.pallas import tpu as pltpu

_E = 3200000
_RI = _E // 64
_RV = _E // 128
_G = 125
_BI = _RI // _G
_BV = _RV // _G


def _iview(a):
    return a.reshape(_RI // 2, 128, 2).swapaxes(1, 2).reshape(_RI, 128)


def _body(m_ref, a1i, a2i, a1v, a2v, oi, ov):
    oi[0] = a1i[...]
    oi[1] = a2i[...] + m_ref[0]
    ov[0] = a1v[...]
    ov[1] = a2v[...]


def kernel(a1_indices, a1_values, a2_indices, a2_values, M):
    idt = a1_indices.dtype
    a1i = _iview(a1_indices)
    a2i = _iview(a2_indices)
    a1v = a1_values.reshape(_RV, 128)
    a2v = a2_values.reshape(_RV, 128)
    m = jnp.asarray(M, idt).reshape(1)

    oi, ov = pl.pallas_call(
        _body,
        grid=(_G,),
        in_specs=[
            pl.BlockSpec(memory_space=pltpu.SMEM),
            pl.BlockSpec((_BI, 128), lambda i: (i, 0)),
            pl.BlockSpec((_BI, 128), lambda i: (i, 0)),
            pl.BlockSpec((_BV, 128), lambda i: (i, 0)),
            pl.BlockSpec((_BV, 128), lambda i: (i, 0)),
        ],
        out_specs=[
            pl.BlockSpec((2, _BI, 128), lambda i: (0, i, 0)),
            pl.BlockSpec((2, _BV, 128), lambda i: (0, i, 0)),
        ],
        out_shape=[
            jax.ShapeDtypeStruct((2, _RI, 128), idt),
            jax.ShapeDtypeStruct((2, _RV, 128), a1_values.dtype),
        ],
    )(m, a1i, a2i, a1v, a2v)

    new_inds = (oi.reshape(2 * _RI // 2, 2, 128)
                  .swapaxes(1, 2)
                  .reshape(2 * _E, 2))
    new_vals = ov.reshape(2 * _E)
    return new_inds, new_vals


# speedup vs baseline: 2.0214x; 2.0214x over previous
import jax
import jax.numpy as jnp
from jax.experimental import pallas as pl
from jax.experimental.pallas import tpu as pltpu

_E = 3200000
_RI = _E // 64
_RV = _E // 128
_G = 5
_BI = _RI // _G
_BV = _RV // _G


def _iview(a):
    return a.reshape(_RI // 2, 128, 2).swapaxes(1, 2).reshape(_RI, 128)


def _body(m_ref, a1i, a2i, a1v, a2v, oi, ov):
    oi[0] = a1i[...]
    oi[1] = a2i[...] + m_ref[0]
    ov[0] = a1v[...]
    ov[1] = a2v[...]


def kernel(a1_indices, a1_values, a2_indices, a2_values, M):
    idt = a1_indices.dtype
    a1i = _iview(a1_indices)
    a2i = _iview(a2_indices)
    a1v = a1_values.reshape(_RV, 128)
    a2v = a2_values.reshape(_RV, 128)
    m = jnp.asarray(M, idt).reshape(1)

    oi, ov = pl.pallas_call(
        _body,
        grid=(_G,),
        in_specs=[
            pl.BlockSpec(memory_space=pltpu.SMEM),
            pl.BlockSpec((_BI, 128), lambda i: (i, 0)),
            pl.BlockSpec((_BI, 128), lambda i: (i, 0)),
            pl.BlockSpec((_BV, 128), lambda i: (i, 0)),
            pl.BlockSpec((_BV, 128), lambda i: (i, 0)),
        ],
        out_specs=[
            pl.BlockSpec((2, _BI, 128), lambda i: (0, i, 0)),
            pl.BlockSpec((2, _BV, 128), lambda i: (0, i, 0)),
        ],
        out_shape=[
            jax.ShapeDtypeStruct((2, _RI, 128), idt),
            jax.ShapeDtypeStruct((2, _RV, 128), a1_values.dtype),
        ],
    )(m, a1i, a2i, a1v, a2v)

    new_inds = (oi.reshape(2 * _RI // 2, 2, 128)
                  .swapaxes(1, 2)
                  .reshape(2 * _E, 2))
    new_vals = ov.reshape(2 * _E)
    return new_inds, new_vals
